# baseline (speedup 1.0000x reference)
import functools
import math

import jax
import jax.numpy as jnp
from jax.experimental import pallas as pl
from jax.experimental.pallas import tpu as pltpu

N = 8192
C = 768
L = 2048
KC = 512
DL = 128


def _attn_seg_kernel(x_ref, wcat_ref, num_ref, den_ref):
    xb = x_ref[...]
    qkv = jnp.dot(xb, wcat_ref[...],
                  preferred_element_type=jnp.float32).astype(jnp.bfloat16)
    q16 = qkv[:, :C]
    k16 = qkv[:, C:2 * C]
    v16 = qkv[:, 2 * C:]
    s = jax.lax.dot_general(
        q16, k16, (((1,), (1,)), ((), ())),
        preferred_element_type=jnp.float32)
    e16 = jnp.exp(s).astype(jnp.bfloat16)
    vcat = jnp.concatenate(
        [v16, jnp.ones((L, DL), jnp.bfloat16)], axis=1)
    nd = jnp.dot(e16, vcat, preferred_element_type=jnp.float32)
    num_ref[...] = nd[:, :C]
    den_ref[...] = nd[:, C:]


def _run_segs(xview, Wcat, nseg):
    return pl.pallas_call(
        _attn_seg_kernel,
        grid=(nseg,),
        in_specs=[
            pl.BlockSpec((L, C), lambda j: (j, 0)),
            pl.BlockSpec((C, 3 * C), lambda j: (0, 0)),
        ],
        out_specs=[
            pl.BlockSpec((L, C), lambda j: (j, 0)),
            pl.BlockSpec((L, DL), lambda j: (j, 0)),
        ],
        out_shape=[
            jax.ShapeDtypeStruct((nseg * L, C), jnp.float32),
            jax.ShapeDtypeStruct((nseg * L, DL), jnp.float32),
        ],
    )(xview, Wcat)


def _mix_kernel(n1_ref, n2_ref, n3_ref, d1_ref, d2_ref, d3_ref, out_ref):
    rows = n1_ref.shape[0]
    zc = jnp.zeros((rows, C), jnp.float32)
    zd = jnp.zeros((rows, DL), jnp.float32)
    n2 = n2_ref[...]
    n3 = n3_ref[...]
    ntot = n1_ref[...] + jnp.concatenate(
        [n2[:, :C], zc, n2[:, C:], zc], axis=1) + jnp.concatenate(
        [n3, zc, zc, zc], axis=1)
    d2 = d2_ref[...]
    d3 = d3_ref[...]
    dtot = d1_ref[...] + jnp.concatenate(
        [d2[:, :DL], zd, d2[:, DL:], zd], axis=1) + jnp.concatenate(
        [d3, zd, zd, zd], axis=1)
    pieces = [
        ntot[:, g * C:(g + 1) * C] / dtot[:, g * DL:g * DL + 1]
        for g in range(4)
    ]
    out_ref[...] = jnp.concatenate(pieces, axis=1)


def _mix(num1, den1, num2, den2, num3, den3):
    nq = N // 4
    rb = nq // 4
    out = pl.pallas_call(
        _mix_kernel,
        grid=(4,),
        in_specs=[
            pl.BlockSpec((rb, 4 * C), lambda j: (j, 0)),
            pl.BlockSpec((rb, 2 * C), lambda j: (j, 0)),
            pl.BlockSpec((rb, C), lambda j: (j, 0)),
            pl.BlockSpec((rb, 4 * DL), lambda j: (j, 0)),
            pl.BlockSpec((rb, 2 * DL), lambda j: (j, 0)),
            pl.BlockSpec((rb, DL), lambda j: (j, 0)),
        ],
        out_specs=pl.BlockSpec((rb, 4 * C), lambda j: (j, 0)),
        out_shape=jax.ShapeDtypeStruct((nq, 4 * C), jnp.float32),
    )(
        num1.reshape(nq, 4 * C),
        num2.reshape(nq, 2 * C),
        num3,
        den1.reshape(nq, 4 * DL),
        den2.reshape(nq, 2 * DL),
        den3,
    )
    return out


def kernel(x, Wq, Wk, Wv):
    x2d = x[0].astype(jnp.bfloat16)
    scale = 1.0 / math.sqrt(C)
    Wcat = jnp.concatenate([Wq * scale, Wk, Wv], axis=1).astype(jnp.bfloat16)
    num1, den1 = _run_segs(x2d, Wcat, 4)
    num2, den2 = _run_segs(x2d.reshape(N // 2, 2 * C), Wcat, 2)
    num3, den3 = _run_segs(x2d.reshape(N // 4, 4 * C), Wcat, 1)
    out = _mix(num1, den1, num2, den2, num3, den3)
    return out.reshape(1, N, C)


# speedup vs baseline: 1.0626x; 1.0626x over previous
import functools
import math

import jax
import jax.numpy as jnp
from jax.experimental import pallas as pl
from jax.experimental.pallas import tpu as pltpu

N = 8192
C = 768
L = 2048
DL = 128


def _attn_kernel(x1_ref, x2_ref, x3_ref, wcat_ref, num_ref, den_ref):
    j = pl.program_id(0)
    branch = (j >= 4).astype(jnp.int32) + (j >= 6).astype(jnp.int32)
    xb = jax.lax.switch(
        branch,
        [lambda: x1_ref[...], lambda: x2_ref[...], lambda: x3_ref[...]],
    )
    wcat = wcat_ref[...]
    q16 = jnp.dot(xb, wcat[:, :C],
                  preferred_element_type=jnp.float32).astype(jnp.bfloat16)
    k16 = jnp.dot(xb, wcat[:, C:2 * C],
                  preferred_element_type=jnp.float32).astype(jnp.bfloat16)
    v16 = jnp.dot(xb, wcat[:, 2 * C:],
                  preferred_element_type=jnp.float32).astype(jnp.bfloat16)
    H = L // 2
    e_halves = []
    for h in range(2):
        s = jax.lax.dot_general(
            q16[h * H:(h + 1) * H, :], k16, (((1,), (1,)), ((), ())),
            preferred_element_type=jnp.float32)
        e_halves.append(jnp.exp(s).astype(jnp.bfloat16))
    vcat = jnp.concatenate(
        [v16, jnp.ones((L, DL), jnp.bfloat16)], axis=1)
    for h in range(2):
        nd = jnp.dot(e_halves[h], vcat, preferred_element_type=jnp.float32)
        nd16 = nd.astype(jnp.bfloat16)
        num_ref[h * H:(h + 1) * H, :] = nd16[:, :C]
        den_ref[h * H:(h + 1) * H, :] = nd16[:, C:]


def _run_segs(x1, x2, x3, Wcat):
    return pl.pallas_call(
        _attn_kernel,
        grid=(7,),
        in_specs=[
            pl.BlockSpec((L, C), lambda j: (jnp.minimum(j, 3), 0)),
            pl.BlockSpec((L, C), lambda j: (jnp.clip(j - 4, 0, 1), 0)),
            pl.BlockSpec((L, C), lambda j: (0, 0)),
            pl.BlockSpec((C, 3 * C), lambda j: (0, 0)),
        ],
        out_specs=[
            pl.BlockSpec((L, C), lambda j: (j, 0)),
            pl.BlockSpec((L, DL), lambda j: (j, 0)),
        ],
        out_shape=[
            jax.ShapeDtypeStruct((7 * L, C), jnp.bfloat16),
            jax.ShapeDtypeStruct((7 * L, DL), jnp.bfloat16),
        ],
    )(x1, x2, x3, Wcat)


def _mix_kernel(n1_ref, n2_ref, n3_ref, d1_ref, d2_ref, d3_ref, out_ref):
    rows = n1_ref.shape[0]
    zc = jnp.zeros((rows, C), jnp.float32)
    zd = jnp.zeros((rows, DL), jnp.float32)
    n2 = n2_ref[...].astype(jnp.float32)
    n3 = n3_ref[...].astype(jnp.float32)
    ntot = n1_ref[...].astype(jnp.float32) + jnp.concatenate(
        [n2[:, :C], zc, n2[:, C:], zc], axis=1) + jnp.concatenate(
        [n3, zc, zc, zc], axis=1)
    d2 = d2_ref[...].astype(jnp.float32)
    d3 = d3_ref[...].astype(jnp.float32)
    dtot = d1_ref[...].astype(jnp.float32) + jnp.concatenate(
        [d2[:, :DL], zd, d2[:, DL:], zd], axis=1) + jnp.concatenate(
        [d3, zd, zd, zd], axis=1)
    pieces = [
        ntot[:, g * C:(g + 1) * C] / dtot[:, g * DL:g * DL + 1]
        for g in range(4)
    ]
    out_ref[...] = jnp.concatenate(pieces, axis=1)


def _mix(num, den):
    nq = N // 4
    rb = nq // 4
    out = pl.pallas_call(
        _mix_kernel,
        grid=(4,),
        in_specs=[
            pl.BlockSpec((rb, 4 * C), lambda j: (j, 0)),
            pl.BlockSpec((rb, 2 * C), lambda j: (j, 0)),
            pl.BlockSpec((rb, C), lambda j: (j, 0)),
            pl.BlockSpec((rb, 4 * DL), lambda j: (j, 0)),
            pl.BlockSpec((rb, 2 * DL), lambda j: (j, 0)),
            pl.BlockSpec((rb, DL), lambda j: (j, 0)),
        ],
        out_specs=pl.BlockSpec((rb, 4 * C), lambda j: (j, 0)),
        out_shape=jax.ShapeDtypeStruct((nq, 4 * C), jnp.float32),
    )(
        num[:4 * L].reshape(nq, 4 * C),
        num[4 * L:6 * L].reshape(nq, 2 * C),
        num[6 * L:],
        den[:4 * L].reshape(nq, 4 * DL),
        den[4 * L:6 * L].reshape(nq, 2 * DL),
        den[6 * L:],
    )
    return out


def kernel(x, Wq, Wk, Wv):
    x2d = x[0].astype(jnp.bfloat16)
    scale = 1.0 / math.sqrt(C)
    Wcat = jnp.concatenate([Wq * scale, Wk, Wv], axis=1).astype(jnp.bfloat16)
    num, den = _run_segs(
        x2d,
        x2d.reshape(N // 2, 2 * C),
        x2d.reshape(N // 4, 4 * C),
        Wcat,
    )
    out = _mix(num, den)
    return out.reshape(1, N, C)


# speedup vs baseline: 1.4769x; 1.3899x over previous
import functools
import math

import jax
import jax.numpy as jnp
from jax.experimental import pallas as pl
from jax.experimental.pallas import tpu as pltpu

N = 8192
C = 768
L = 2048
DL = 128


def _attn_kernel(x_ref, wcat_ref, num_ref, den_ref, xs_ref, sem):
    j = pl.program_id(0)

    @pl.when(j == 0)
    def _load_x():
        cp = pltpu.make_async_copy(x_ref, xs_ref, sem)
        cp.start()
        cp.wait()

    def _stride1():
        return xs_ref[pl.ds(L * j, L), :]

    def _stride2():
        w = xs_ref[pl.ds(2 * L * (j - 4), 2 * L), :]
        return w.reshape(L, 2 * C)[:, :C]

    def _stride4():
        return xs_ref[...].reshape(L, 4 * C)[:, :C]

    branch = (j >= 4).astype(jnp.int32) + (j >= 6).astype(jnp.int32)
    xb = jax.lax.switch(branch, [_stride1, _stride2, _stride4])

    wcat = wcat_ref[...]
    q16 = jnp.dot(xb, wcat[:, :C],
                  preferred_element_type=jnp.float32).astype(jnp.bfloat16)
    k16 = jnp.dot(xb, wcat[:, C:2 * C],
                  preferred_element_type=jnp.float32).astype(jnp.bfloat16)
    v16 = jnp.dot(xb, wcat[:, 2 * C:],
                  preferred_element_type=jnp.float32).astype(jnp.bfloat16)
    H = L // 2
    e_halves = []
    for h in range(2):
        s = jax.lax.dot_general(
            q16[h * H:(h + 1) * H, :], k16, (((1,), (1,)), ((), ())),
            preferred_element_type=jnp.float32)
        e_halves.append(jnp.exp(s).astype(jnp.bfloat16))
    vcat = jnp.concatenate(
        [v16, jnp.ones((L, DL), jnp.bfloat16)], axis=1)
    for h in range(2):
        nd = jnp.dot(e_halves[h], vcat, preferred_element_type=jnp.float32)
        nd16 = nd.astype(jnp.bfloat16)
        num_ref[h * H:(h + 1) * H, :] = nd16[:, :C]
        den_ref[h * H:(h + 1) * H, :] = nd16[:, C:]


def _run_segs(x16, Wcat):
    return pl.pallas_call(
        _attn_kernel,
        grid=(7,),
        in_specs=[
            pl.BlockSpec(memory_space=pl.ANY),
            pl.BlockSpec((C, 3 * C), lambda j: (0, 0)),
        ],
        out_specs=[
            pl.BlockSpec((L, C), lambda j: (j, 0)),
            pl.BlockSpec((L, DL), lambda j: (j, 0)),
        ],
        out_shape=[
            jax.ShapeDtypeStruct((7 * L, C), jnp.bfloat16),
            jax.ShapeDtypeStruct((7 * L, DL), jnp.bfloat16),
        ],
        scratch_shapes=[
            pltpu.VMEM((N, C), jnp.bfloat16),
            pltpu.SemaphoreType.DMA,
        ],
    )(x16, Wcat)


def _mix_kernel(n1_ref, n2_ref, n3_ref, d1_ref, d2_ref, d3_ref, out_ref):
    R = L // 4
    n1q = n1_ref[...].astype(jnp.float32).reshape(R, 4 * C)
    n2p = n2_ref[...].astype(jnp.float32).reshape(R, 2 * C)
    n3 = n3_ref[...].astype(jnp.float32)
    zc = jnp.zeros((R, C), jnp.float32)
    ntot = n1q + jnp.concatenate(
        [n2p[:, :C], zc, n2p[:, C:], zc], axis=1) + jnp.concatenate(
        [n3, zc, zc, zc], axis=1)
    d1q = d1_ref[...].astype(jnp.float32).reshape(R, 4 * DL)
    d2p = d2_ref[...].astype(jnp.float32).reshape(R, 2 * DL)
    d3 = d3_ref[...].astype(jnp.float32)
    zd = jnp.zeros((R, DL), jnp.float32)
    dtot = d1q + jnp.concatenate(
        [d2p[:, :DL], zd, d2p[:, DL:], zd], axis=1) + jnp.concatenate(
        [d3, zd, zd, zd], axis=1)
    pieces = [
        ntot[:, g * C:(g + 1) * C] / dtot[:, g * DL:g * DL + 1]
        for g in range(4)
    ]
    out_ref[...] = jnp.concatenate(pieces, axis=1).reshape(L, C)


def _mix(num, den):
    return pl.pallas_call(
        _mix_kernel,
        grid=(4,),
        in_specs=[
            pl.BlockSpec((L, C), lambda j: (j, 0)),
            pl.BlockSpec((L // 2, C), lambda j: (8 + j, 0)),
            pl.BlockSpec((L // 4, C), lambda j: (24 + j, 0)),
            pl.BlockSpec((L, DL), lambda j: (j, 0)),
            pl.BlockSpec((L // 2, DL), lambda j: (8 + j, 0)),
            pl.BlockSpec((L // 4, DL), lambda j: (24 + j, 0)),
        ],
        out_specs=pl.BlockSpec((L, C), lambda j: (j, 0)),
        out_shape=jax.ShapeDtypeStruct((N, C), jnp.float32),
    )(num, num, num, den, den, den)


def kernel(x, Wq, Wk, Wv):
    x16 = x[0].astype(jnp.bfloat16)
    scale = 1.0 / math.sqrt(C)
    Wcat = jnp.concatenate([Wq * scale, Wk, Wv], axis=1).astype(jnp.bfloat16)
    num, den = _run_segs(x16, Wcat)
    out = _mix(num, den)
    return out.reshape(1, N, C)
